# baseline (speedup 1.0000x reference)
import numpy as np
import jax
import jax.numpy as jnp
from jax import lax
from jax.experimental import pallas as pl
from jax.experimental.pallas import tpu as pltpu
from jax.experimental.pallas import tpu_sc as plsc

_BS = 16
_T = 30
_TP = 32
_G = 19
_NB = 5
_NC = 20
_P = _G * _G
_NCH = _NC + 5

_ANCW = np.array([1.3221, 3.19275, 5.05587, 9.47112, 11.2364], dtype=np.float32)
_ANCH = np.array([1.73145, 4.00944, 8.09892, 4.84053, 10.0071], dtype=np.float32)
_AWG = (_ANCW.astype(np.float64) / _G).astype(np.float32)
_AHG = (_ANCH.astype(np.float64) / _G).astype(np.float32)

_OBJECT_SCALE = 5.0
_NOOBJECT_SCALE = 1.0
_CLASS_SCALE = 1.0
_COORD_SCALE = 1.0
_PRIOR_SCALE = 0.01
_IOU_THRESH = 0.6

_XYF = np.stack([np.arange(_P, dtype=np.float32) % _G,
                 np.arange(_P, dtype=np.float32) // _G]).reshape(2, 1, _P)
_TRIU = (np.arange(_TP)[:, None] <= np.arange(_TP)[None, :]).astype(np.float32)
_LT = (np.arange(_TP)[None, :] < np.arange(_TP)[:, None]).astype(np.float32)
_MKS = np.stack([_TRIU, _LT])
_AW4 = np.stack([_ANCW, _ANCH, _AWG, _AHG]).reshape(4, _NB, 1)


def _tc_kernel(p_ref, g_ref, t_ref, xy_ref, aw_ref, mk_ref, o_ref):
    g = jnp.float32(_G)
    tg = t_ref[...]
    bx = tg[1]
    by = tg[2]
    bw = tg[3]
    bh = tg[4]
    xy = xy_ref[...]
    xf = xy[0]
    yf = xy[1]
    triu = mk_ref[0]

    zf = (bx == 0.0).astype(jnp.float32)
    zcount = jax.lax.dot_general(
        zf, triu, (((1,), (0,)), ((), ())),
        preferred_element_type=jnp.float32)
    valid = zcount == 0.0

    bx1 = bx - bw * 0.5
    bx2 = bx + bw * 0.5
    by1 = by - bh * 0.5
    by2 = by + bh * 0.5
    a2s = jnp.where(valid, bw * bh * _IOU_THRESH, jnp.float32(np.inf))
    a2s6 = a2s * 0.625

    s_noobj = jnp.float32(0.0)
    s_prior = jnp.float32(0.0)
    for a in range(_NB):
        r0 = a * _NCH
        tx = jax.nn.sigmoid(p_ref[:, r0 + 0, :])
        ty = jax.nn.sigmoid(p_ref[:, r0 + 1, :])
        tw = p_ref[:, r0 + 2, :]
        th = p_ref[:, r0 + 3, :]
        conf = jax.nn.sigmoid(p_ref[:, r0 + 4, :])
        s_prior = s_prior + jnp.sum(
            (tx - 0.5) ** 2 + (ty - 0.5) ** 2 + tw * tw + th * th)

        cw = jnp.exp(tw) * float(_AWG[a])
        ch = jnp.exp(th) * float(_AHG[a])
        cx = (tx + xf) / g
        cy = (ty + yf) / g
        cx1 = cx - cw * 0.5
        cx2 = cx + cw * 0.5
        cy1 = cy - ch * 0.5
        cy2 = cy + ch * 0.5
        a1s6 = (cw * ch) * (_IOU_THRESH * 0.625)

        any_over = jnp.zeros((_BS, _P), dtype=jnp.bool_)
        for t in range(_T):
            iw = jnp.maximum(
                jnp.minimum(cx2, bx2[:, t:t + 1]) - jnp.maximum(cx1, bx1[:, t:t + 1]),
                0.0)
            ih = (jnp.minimum(cy2, by2[:, t:t + 1])
                  - jnp.maximum(cy1, by1[:, t:t + 1]))
            over = iw * ih > a1s6 + a2s6[:, t:t + 1]
            any_over = jnp.logical_or(any_over, over)
        nof = 1.0 - any_over.astype(jnp.float32)

        s_noobj = s_noobj + jnp.sum(conf * conf * nof)

    dense_total = _NOOBJECT_SCALE * s_noobj + _PRIOR_SCALE * s_prior

    corr_total = _corr_math(g_ref, t_ref, aw_ref, mk_ref)
    o_ref[...] = jnp.reshape((dense_total + corr_total) / jnp.float32(_BS), (1, 1))


def _sc_gather_body(pf_ref, t_ref, out_ref, tgv, idxv, gatv, sem):
    g = 19.0
    cc = lax.axis_index("c")
    ss = lax.axis_index("s")
    wid = ss * 2 + cc

    @pl.when(wid < _BS)
    def _():
        b = wid
        pltpu.sync_copy(t_ref.at[1, b], tgv.at[0])
        pltpu.sync_copy(t_ref.at[2, b], tgv.at[1])
        pltpu.sync_copy(t_ref.at[3, b], tgv.at[2])
        pltpu.sync_copy(t_ref.at[4, b], tgv.at[3])
        for gc in range(2):
            xs = tgv[0, pl.ds(gc * 16, 16)]
            ys = tgv[1, pl.ds(gc * 16, 16)]
            ws = tgv[2, pl.ds(gc * 16, 16)]
            hs = tgv[3, pl.ds(gc * 16, 16)]
            iv = (xs * g).astype(jnp.int32)
            jv = (ys * g).astype(jnp.int32)
            area = ws * hs
            rbest = None
            asel = jnp.zeros((16,), jnp.int32)
            for a in range(_NB):
                aw = float(_AWG[a])
                ah = float(_AHG[a])
                inter = jnp.minimum(aw, ws) * jnp.minimum(ah, hs)
                r = inter / (aw * ah + area - inter)
                if rbest is None:
                    rbest = r
                else:
                    upd = r > rbest
                    asel = jnp.where(upd, a, asel)
                    rbest = jnp.where(upd, r, rbest)
            base = (b * (_NB * _NCH) + asel * _NCH) * _P + jv * _G + iv
            for f in range(28):
                fc = min(f, _NCH - 1)
                pos = f * 32 + gc * 16
                idxv[pos // 128, pl.ds(pos % 128, 16)] = base + fc * _P
            gatv[7, pl.ds(gc * 16, 16)] = asel.astype(jnp.float32)
            gatv[7, pl.ds(32 + gc * 16, 16)] = iv.astype(jnp.float32)
            gatv[7, pl.ds(64 + gc * 16, 16)] = jv.astype(jnp.float32)
        cps = [pltpu.async_copy(pf_ref.at[idxv.at[ci]], gatv.at[ci], sem)
               for ci in range(7)]
        for cp in cps:
            cp.wait()
        pltpu.sync_copy(gatv, out_ref.at[b])


def _corr_math(g_ref, t_ref, aw_ref, mk_ref):
    g = jnp.float32(_G)
    tg = t_ref[...]
    kls = tg[0]
    bx = tg[1]
    by = tg[2]
    bw = tg[3]
    bh = tg[4]
    awh = aw_ref[...]
    awc = awh[0:1].reshape(1, _NB, 1)
    ahc = awh[1:2].reshape(1, _NB, 1)
    awgc = awh[2:3].reshape(1, _NB, 1)
    ahgc = awh[3:4].reshape(1, _NB, 1)
    triu = mk_ref[0]
    ltm = mk_ref[1]

    zf = (bx == 0.0).astype(jnp.float32)
    zcount = jax.lax.dot_general(
        zf, triu, (((1,), (0,)), ((), ())),
        preferred_element_type=jnp.float32)
    valid = zcount == 0.0
    validf = valid.astype(jnp.float32)

    garr = g_ref[...]

    def slot(f):
        return garr[:, f // 4, (f % 4) * 32:(f % 4) * 32 + 32]

    tx_c = jax.nn.sigmoid(slot(0))
    ty_c = jax.nn.sigmoid(slot(1))
    tw_c = slot(2)
    th_c = slot(3)
    conf_c = jax.nn.sigmoid(slot(4))
    logits_c = jnp.concatenate(
        [slot(5 + q)[:, None, :] for q in range(_NC)], axis=1)
    asel = garr[:, 7, 0:32]
    i_f = garr[:, 7, 32:64]
    j_f = garr[:, 7, 64:96]

    aiota = jax.lax.broadcasted_iota(jnp.int32, (_BS, _NB, _TP), 1).astype(jnp.float32)
    onehot_a = (aiota == asel[:, None, :]).astype(jnp.float32)
    acw = jnp.sum(onehot_a * awc, axis=1)
    ach = jnp.sum(onehot_a * ahc, axis=1)
    awgs = jnp.sum(onehot_a * awgc, axis=1)
    ahgs = jnp.sum(onehot_a * ahgc, axis=1)

    c_cell = asel * jnp.float32(_P) + j_f * g + i_f
    same = c_cell[:, :, None] == c_cell[:, None, :]
    prev = ltm[None, :, :] * validf[:, None, :]
    blocked = jnp.max(same.astype(jnp.float32) * prev, axis=2) > 0.0
    applied = jnp.logical_and(valid, jnp.logical_not(blocked))
    appliedf = applied.astype(jnp.float32)

    pcw = jnp.exp(tw_c) * awgs
    pch = jnp.exp(th_c) * ahgs
    pcx = (tx_c + i_f) / g
    pcy = (ty_c + j_f) / g
    p_x1 = pcx - pcw * 0.5
    p_x2 = pcx + pcw * 0.5
    p_y1 = pcy - pch * 0.5
    p_y2 = pcy + pch * 0.5

    bx1 = bx - bw * 0.5
    bx2 = bx + bw * 0.5
    by1 = by - bh * 0.5
    by2 = by + bh * 0.5
    a2s = jnp.where(valid, bw * bh * _IOU_THRESH, jnp.float32(np.inf))
    a2s6 = a2s * 0.625
    a1s6 = (pcw * pch) * (_IOU_THRESH * 0.625)

    any_over = jnp.zeros((_BS, _TP), dtype=jnp.bool_)
    for t in range(_T):
        iw = jnp.maximum(
            jnp.minimum(p_x2, bx2[:, t:t + 1]) - jnp.maximum(p_x1, bx1[:, t:t + 1]),
            0.0)
        ih = (jnp.minimum(p_y2, by2[:, t:t + 1])
              - jnp.maximum(p_y1, by1[:, t:t + 1]))
        over = iw * ih > a1s6 + a2s6[:, t:t + 1]
        any_over = jnp.logical_or(any_over, over)
    nof_c = 1.0 - any_over.astype(jnp.float32)

    iw = jnp.maximum(jnp.minimum(p_x2, bx2) - jnp.maximum(p_x1, bx1), 0.0)
    ih = jnp.maximum(jnp.minimum(p_y2, by2) - jnp.maximum(p_y1, by1), 0.0)
    inter = iw * ih
    union = (p_x2 - p_x1) * (p_y2 - p_y1) + (bx2 - bx1) * (by2 - by1) - inter
    iou_real = inter / union

    ex = bx * g - i_f
    ey = by * g - j_f
    ew = jnp.log(bw * g / acw)
    eh = jnp.log(bh * g / ach)
    sc = 2.0 - bw * bh
    coord_sum = ((tx_c - ex) ** 2 + (ty_c - ey) ** 2
                 + (tw_c - ew) ** 2 + (th_c - eh) ** 2) * (sc * sc)

    m = jnp.max(logits_c, axis=1, keepdims=True)
    e = jnp.exp(logits_c - m)
    cls_prob = e / jnp.sum(e, axis=1, keepdims=True)
    kiota = jax.lax.broadcasted_iota(jnp.int32, (_BS, _NC, _TP), 1).astype(jnp.float32)
    oh_k = (kiota == jnp.floor(kls)[:, None, :]).astype(jnp.float32)
    cls_sum = jnp.sum((cls_prob - oh_k) ** 2, axis=1)

    prior_sum = (tx_c - 0.5) ** 2 + (ty_c - 0.5) ** 2 + tw_c * tw_c + th_c * th_c

    delta = (_OBJECT_SCALE * (conf_c - iou_real) ** 2
             + _CLASS_SCALE * cls_sum
             + _COORD_SCALE * coord_sum
             - _NOOBJECT_SCALE * conf_c * conf_c * nof_c
             - _PRIOR_SCALE * prior_sum)

    return jnp.sum(appliedf * delta)


def kernel(preds, targets):
    p3 = preds.reshape(_BS, _NB * _NCH, _P)
    pflat = preds.reshape(_BS * _NB * _NCH * _P)
    tgp = jnp.pad(jnp.transpose(targets, (2, 0, 1)), ((0, 0), (0, 0), (0, _TP - _T)))

    mesh = plsc.VectorSubcoreMesh(core_axis_name="c", subcore_axis_name="s")
    gathered = pl.kernel(
        _sc_gather_body,
        out_type=jax.ShapeDtypeStruct((_BS, 8, 128), jnp.float32),
        mesh=mesh,
        scratch_types=[
            pltpu.VMEM((4, _TP), jnp.float32),
            pltpu.VMEM((7, 128), jnp.int32),
            pltpu.VMEM((8, 128), jnp.float32),
            pltpu.SemaphoreType.DMA,
        ],
    )(pflat, tgp)

    out = pl.pallas_call(
        _tc_kernel,
        out_shape=jax.ShapeDtypeStruct((1, 1), jnp.float32),
    )(p3, gathered, tgp, jnp.asarray(_XYF), jnp.asarray(_AW4), jnp.asarray(_MKS))

    return out[0, 0]
